# Initial kernel scaffold; baseline (speedup 1.0000x reference)
import jax
import jax.numpy as jnp
from jax.experimental import pallas as pl


def kernel(embeds, labels, table, W, b, neg_samples):
    raise NotImplementedError("write your pallas kernel here")


# speedup vs baseline: 27.1646x; 27.1646x over previous
import jax
import jax.numpy as jnp
from jax import lax
from jax.experimental import pallas as pl
from jax.experimental.pallas import tpu as pltpu
from jax.experimental.pallas import tpu_sc as plsc

_TEMP = 1.0


def _proj_body(e_ref, wt_ref, b_ref, o_ref):
    o_ref[...] = (
        jnp.dot(e_ref[...], wt_ref[...], preferred_element_type=jnp.float32)
        + b_ref[...]
    )


def _project(e2, Wt, b2, N, D):
    rows = 2048
    return pl.pallas_call(
        _proj_body,
        grid=(N // rows,),
        in_specs=[
            pl.BlockSpec((rows, D), lambda i: (i, 0)),
            pl.BlockSpec((D, D), lambda i: (0, 0)),
            pl.BlockSpec((1, D), lambda i: (0, 0)),
        ],
        out_specs=pl.BlockSpec((rows, D), lambda i: (i, 0)),
        out_shape=jax.ShapeDtypeStruct((N, D), jnp.float32),
    )(e2, Wt, b2)


def _make_scores(N, D, NEG):
    NC, NS = 2, 16
    NW = NC * NS
    RW = N // NW
    R = 8
    NCH = RW // R
    mesh = plsc.VectorSubcoreMesh(
        core_axis_name="c", subcore_axis_name="s", num_cores=NC, num_subcores=NS
    )

    L = 16
    JG = NEG // L

    def body(x_hbm, lab_hbm, idx_hbm, table_hbm, pos_hbm, neg_hbm,
             idx_all, lab_all, x_v, pos_v, neg_v, nout_v, pout_v,
             sem_in0, sem_in1, sem_out0, sem_out1):
        wid = lax.axis_index("s") * NC + lax.axis_index("c")
        base = wid * RW
        pltpu.sync_copy(idx_hbm.at[pl.ds(base, RW)], idx_all)
        pltpu.sync_copy(lab_hbm.at[pl.ds(base, RW)], lab_all)

        sems_in = (sem_in0, sem_in1)
        sems_out = (sem_out0, sem_out1)
        lane = jnp.arange(L, dtype=jnp.int32)

        def in_copies(gg, b):
            lr = gg * R
            sem = sems_in[b]
            cps = [
                pltpu.make_async_copy(
                    x_hbm.at[pl.ds(base + lr, R)], x_v.at[b], sem),
                pltpu.make_async_copy(
                    table_hbm.at[lab_all.at[pl.ds(lr, R)]], pos_v.at[b], sem),
            ]
            for r in range(R):
                cps.append(pltpu.make_async_copy(
                    table_hbm.at[idx_all.at[lr + r]], neg_v.at[b, r], sem))
            return cps

        def out_copies(gg, b):
            return [
                pltpu.make_async_copy(
                    nout_v.at[b], neg_hbm.at[pl.ds(base + gg * R, R)],
                    sems_out[b]),
                pltpu.make_async_copy(
                    pout_v.at[b], pos_hbm.at[wid * NCH + gg], sems_out[b]),
            ]

        def fire(gg, b):
            for c in in_copies(gg, b):
                c.start()

        fire(0, 0)
        fire(1, 1)

        def tbody(t, carry):
            for b in range(2):
                gg = t * 2 + b
                for c in in_copies(gg, b):
                    c.wait()

                @pl.when(gg >= 2)
                def _():
                    for c in out_copies(gg - 2, b):
                        c.wait()

                def rbody(r, pos_acc, b=b):
                    x0 = x_v[b, r, 0:16]
                    x1 = x_v[b, r, 16:32]
                    x2 = x_v[b, r, 32:48]
                    x3 = x_v[b, r, 48:64]
                    ps = (pos_v[b, r, 0:16] * x0 + pos_v[b, r, 16:32] * x1
                          + pos_v[b, r, 32:48] * x2 + pos_v[b, r, 48:64] * x3)
                    pos_acc = pos_acc + jnp.sum(ps) * jnp.where(
                        lane == r, 1.0, 0.0).astype(jnp.float32)

                    def gbody(jg, c2, b=b, r=r, x0=x0, x1=x1, x2=x2, x3=x3):
                        acc = jnp.zeros((L,), jnp.float32)
                        for jj in range(L):
                            j = jg * L + jj
                            a = (neg_v[b, r, j, 0:16] * x0
                                 + neg_v[b, r, j, 16:32] * x1
                                 + neg_v[b, r, j, 32:48] * x2
                                 + neg_v[b, r, j, 48:64] * x3)
                            oh = jnp.where(lane == jj, 1.0, 0.0).astype(
                                jnp.float32)
                            acc = acc + jnp.sum(a) * oh
                        nout_v[b, r, pl.ds(jg * L, L)] = acc
                        return c2

                    lax.fori_loop(0, JG, gbody, 0)
                    return pos_acc

                pos_acc = lax.fori_loop(
                    0, R, rbody, jnp.zeros((L,), jnp.float32))
                pout_v[b, :] = pos_acc

                for c in out_copies(gg, b):
                    c.start()

                @pl.when(gg + 2 < NCH)
                def _():
                    fire(gg + 2, b)
            return carry

        lax.fori_loop(0, NCH // 2, tbody, 0)
        for c in out_copies(NCH - 2, 0):
            c.wait()
        for c in out_copies(NCH - 1, 1):
            c.wait()

    return pl.kernel(
        body,
        out_type=[
            jax.ShapeDtypeStruct((NW * NCH, L), jnp.float32),
            jax.ShapeDtypeStruct((N, NEG), jnp.float32),
        ],
        mesh=mesh,
        compiler_params=pltpu.CompilerParams(
            needs_layout_passes=False, use_tc_tiling_on_sc=False),
        scratch_types=[
            pltpu.VMEM((RW, NEG), jnp.int32),
            pltpu.VMEM((RW,), jnp.int32),
            pltpu.VMEM((2, R, D), jnp.float32),
            pltpu.VMEM((2, R, D), jnp.float32),
            pltpu.VMEM((2, R, NEG, D), jnp.float32),
            pltpu.VMEM((2, R, NEG), jnp.float32),
            pltpu.VMEM((2, L), jnp.float32),
            pltpu.SemaphoreType.DMA,
            pltpu.SemaphoreType.DMA,
            pltpu.SemaphoreType.DMA,
            pltpu.SemaphoreType.DMA,
        ],
    )


def _make_loss(N, NEG):
    rows = 2048
    inv_t = 1.0 / _TEMP

    def body(pos_ref, neg_ref, logits_ref, loss_ref):
        i = pl.program_id(0)
        pos = pos_ref[...]
        neg = neg_ref[...]
        logits_ref[...] = jnp.concatenate([pos, neg], axis=1)
        sp = pos * inv_t
        sn = neg * inv_t
        m = jnp.maximum(sp, jnp.max(sn, axis=1, keepdims=True))
        lse = m + jnp.log(
            jnp.exp(sp - m) + jnp.sum(jnp.exp(sn - m), axis=1, keepdims=True))
        c = jnp.sum(lse - sp)

        @pl.when(i == 0)
        def _():
            loss_ref[0, 0] = 0.0

        loss_ref[0, 0] += c / N

    return pl.pallas_call(
        body,
        grid=(N // rows,),
        in_specs=[
            pl.BlockSpec((rows, 1), lambda i: (i, 0)),
            pl.BlockSpec((rows, NEG), lambda i: (i, 0)),
        ],
        out_specs=[
            pl.BlockSpec((rows, 1 + NEG), lambda i: (i, 0)),
            pl.BlockSpec((1, 1), lambda i: (0, 0), memory_space=pltpu.SMEM),
        ],
        out_shape=[
            jax.ShapeDtypeStruct((N, 1 + NEG), jnp.float32),
            jax.ShapeDtypeStruct((1, 1), jnp.float32),
        ],
    )


def kernel(embeds, labels, table, W, b, neg_samples):
    B, T, D = embeds.shape
    N = B * T
    NEG = neg_samples.shape[1]

    e2 = embeds.reshape(N, D)
    x = _project(e2, W.T, b.reshape(1, D), N, D)
    pos2, neg = _make_scores(N, D, NEG)(
        x, labels.reshape(N), neg_samples, table)
    R = 8
    pos = pos2[:, :R].reshape(N, 1)
    logits, loss = _make_loss(N, NEG)(pos, neg)
    return logits, loss.reshape(())
